# baseline (speedup 1.0000x reference)
import functools

import jax
import jax.numpy as jnp
from jax import lax
from jax.experimental import pallas as pl
from jax.experimental.pallas import tpu as pltpu
from jax.experimental.pallas import tpu_sc as plsc

L = 16
NC = 2
NS = 16
NW = NC * NS
GRP = 16
UB = 8
US = 4
DEPTH = 64
CROWS = DEPTH + L + US
CELLROWS = 64
CCELLS = CELLROWS + UB
PAD = 16 * L
KTOP = 64
MINNORM = 1.1754944e-38


@functools.lru_cache(maxsize=None)
def _build(nrows, ncols):
    assert ncols % (L * GRP) == 0 and nrows % (2 * NW) == 0
    nch = ncols // L
    ngrp = nch // GRP
    rows_per = nrows // NW

    mesh = plsc.VectorSubcoreMesh(
        core_axis_name="c", subcore_axis_name="s",
        num_cores=NC, num_subcores=NS)

    @functools.partial(
        pl.kernel,
        out_type=jax.ShapeDtypeStruct((nrows, ncols), jnp.float32),
        mesh=mesh,
        compiler_params=pltpu.CompilerParams(needs_layout_passes=False),
        scratch_types=[
            pltpu.VMEM((ncols + PAD,), jnp.float32),
            pltpu.VMEM((ncols + PAD,), jnp.float32),
            pltpu.VMEM((ncols,), jnp.float32),
            pltpu.VMEM((ngrp * L,), jnp.float32),
            pltpu.VMEM((CCELLS * L,), jnp.int32),
            pltpu.VMEM((CROWS * L,), jnp.int32),
            pltpu.VMEM((CROWS * L,), jnp.int32),
            pltpu.VMEM((CROWS * L,), jnp.float32),
            pltpu.VMEM((CROWS * L,), jnp.float32),
            pltpu.SemaphoreType.DMA,
            pltpu.SemaphoreType.DMA,
            pltpu.SemaphoreType.DMA,
        ],
    )
    def topk_mask(x_hbm, out_hbm, rb0, rb1, zbuf, gmaxb, cellb,
                  cc0, cc1, cv0, cv1, si0, si1, so):
        wid = lax.axis_index("s") * NC + lax.axis_index("c")
        base = wid * rows_per
        lanes = lax.iota(jnp.int32, L)
        lanes16 = lanes * L
        zf = jnp.zeros((L,), jnp.float32)
        onei = jnp.full((L,), 1, jnp.int32)
        inc16 = jnp.full((L,), 16, jnp.int32)
        zi = jnp.zeros((L,), jnp.int32)
        lane15 = jnp.full((L,), 15, jnp.int32)
        ktopv = jnp.full((L,), KTOP, jnp.int32)
        minnv = jnp.full((L,), MINNORM, jnp.float32)
        posclamp = lanes + (DEPTH - 1) * L
        cellclamp = lanes + (CELLROWS - 1) * L
        sentv = jnp.full((L,), ncols, jnp.int32)

        rbufs = (rb0, rb1)
        ccols = (cc0, cc1)
        cvals = (cv0, cv1)
        sins = (si0, si1)

        def z0_body(g, _):
            for j in range(UB):
                zbuf[pl.ds((g * UB + j) * L, L)] = zf
            return 0

        lax.fori_loop(0, nch // UB, z0_body, 0)
        for j in range(PAD // L):
            rb0[pl.ds(ncols + j * L, L)] = zf
            rb1[pl.ds(ncols + j * L, L)] = zf
        for j in range(CCELLS):
            cellb[pl.ds(j * L, L)] = sentv
        for j in range(CROWS):
            cc0[pl.ds(j * L, L)] = zi
            cc1[pl.ds(j * L, L)] = zi

        pltpu.async_copy(x_hbm.at[base], rb0.at[pl.ds(0, ncols)], si0)

        def process(i, b, tf_prev, niter_prev):
            r = base + 2 * i + b
            rowbuf = rbufs[b]
            ccol = ccols[b]
            cval = cvals[b]
            sin = sins[b]
            nccol = ccols[1 - b]
            ncval = cvals[1 - b]

            if b == 0:
                pltpu.async_copy(x_hbm.at[r + 1], rbufs[1].at[pl.ds(0, ncols)], sins[1])
            else:
                @pl.when(i < rows_per // 2 - 1)
                def _():
                    pltpu.async_copy(x_hbm.at[r + 1], rbufs[0].at[pl.ds(0, ncols)], sins[0])

            pltpu.make_async_copy(x_hbm.at[base], rowbuf.at[pl.ds(0, ncols)], sin).wait()

            def grp_body(g, carry):
                m0, m1, m2, m3 = carry
                gmax = zf
                for j in range(GRP):
                    gmax = jnp.maximum(gmax, rowbuf[pl.ds((g * GRP + j) * L, L)])
                gmaxb[pl.ds(g * L, L)] = gmax
                t0 = jnp.maximum(m0, gmax)
                b0 = jnp.minimum(m0, gmax)
                t1 = jnp.maximum(m1, b0)
                b1 = jnp.minimum(m1, b0)
                t2 = jnp.maximum(m2, b1)
                b2 = jnp.minimum(m2, b1)
                t3 = jnp.maximum(m3, b2)
                return t0, t1, t2, t3

            _, _, _, m3 = lax.fori_loop(0, ngrp, grp_body, (zf, zf, zf, zf))
            mv0 = jnp.maximum(m3, minnv)
            mv = jnp.negative(plsc.cummax(jnp.negative(mv0))).at[lane15].get(
                mode="promise_in_bounds")

            def p1_body(gb, carry):
                coff, gstart = carry
                gs = [gmaxb[pl.ds((gb * UB + j) * L, L)] for j in range(UB)]
                keeps = [g >= mv for g in gs]
                sels = [jnp.where(k, inc16, zi) for k in keeps]
                starts = [gstart + j * (GRP * L) for j in range(UB)]
                for j in range(UB):
                    plsc.store_scatter(cellb, [coff], starts[j], mask=keeps[j])
                    coff = coff + sels[j]
                return jnp.minimum(coff, cellclamp), gstart + UB * (GRP * L)

            coff, _ = lax.fori_loop(0, ngrp // UB, p1_body, (lanes, lanes))
            ncrows = jnp.minimum(jnp.max(coff) >> 4, CELLROWS - 1) + 1

            def p2_body(j, off):
                crow = cellb[pl.ds(j * L, L)]
                for u in range(L):
                    cs = crow.at[jnp.full((L,), u, jnp.int32)].get(
                        mode="promise_in_bounds")
                    idx = cs + lanes16
                    g = plsc.load_gather(rowbuf, [idx])
                    keep = g >= mv
                    plsc.store_scatter(ccol, [off], idx, mask=keep)
                    off = off + jnp.where(keep, inc16, zi)
                return jnp.minimum(off, posclamp)

            off = lax.fori_loop(0, ncrows, p2_body, lanes)
            def cz_body(j, _):
                cellb[pl.ds(j * L, L)] = sentv
                return 0

            lax.fori_loop(0, ncrows, cz_body, 0)

            cnt_vec = lax.shift_right_logical(off - lanes, 4)
            nrows_c = jnp.minimum(jnp.max(off) >> 4, DEPTH - 1) + 1
            niter = (nrows_c + (US - 1)) // US

            def g_body(j, rowv):
                colvs = [ccol[pl.ds((j * US + u) * L, L)] for u in range(US)]
                gs = [plsc.load_gather(rowbuf, [c]) for c in colvs]
                for u in range(US):
                    valid = (rowv + u) < cnt_vec
                    cval[pl.ds((j * US + u) * L, L)] = jnp.where(valid, gs[u], 0.0)
                return rowv + US

            lax.fori_loop(0, niter, g_body, zi)

            def bit_body(bb, carry):
                tbv, bitv = carry
                pv = tbv | bitv
                pfv = plsc.bitcast(pv, jnp.float32)

                def cnt_body(j, acc):
                    cvs = [cval[pl.ds((j * US + u) * L, L)] for u in range(US)]
                    for cv in cvs:
                        acc = acc + jnp.where(cv >= pfv, onei, zi)
                    return acc

                acc = lax.fori_loop(0, niter, cnt_body, zi)
                tot = plsc.cumsum(acc).at[lane15].get(mode="promise_in_bounds")
                tbv = jnp.where(tot >= ktopv, pv, tbv)
                return tbv, lax.shift_right_logical(bitv, 1)

            tbv, _ = lax.fori_loop(
                0, 31, bit_body,
                (zi, jnp.full((L,), 1 << 30, jnp.int32)))
            tfv = plsc.bitcast(jnp.maximum(tbv, onei), jnp.float32)

            if b == 0:
                @pl.when(i >= 1)
                def _():
                    pltpu.make_async_copy(zbuf, out_hbm.at[base], so).wait()
            else:
                pltpu.make_async_copy(zbuf, out_hbm.at[base], so).wait()

            def uz_body(j, _):
                vals = [ncval[pl.ds((j * US + u) * L, L)] for u in range(US)]
                cols = [nccol[pl.ds((j * US + u) * L, L)] for u in range(US)]
                for u in range(US):
                    plsc.store_scatter(zbuf, [cols[u]], zf,
                                       mask=vals[u] >= tf_prev)
                return 0

            lax.fori_loop(0, niter_prev, uz_body, 0)

            def sc_body(j, _):
                vals = [cval[pl.ds((j * US + u) * L, L)] for u in range(US)]
                cols = [ccol[pl.ds((j * US + u) * L, L)] for u in range(US)]
                for u in range(US):
                    plsc.store_scatter(zbuf, [cols[u]], vals[u],
                                       mask=vals[u] >= tfv)
                return 0

            lax.fori_loop(0, niter, sc_body, 0)
            pltpu.async_copy(zbuf, out_hbm.at[r], so)
            return tfv, niter

        def row_pair(i, carry):
            tf_prev, niter_prev = carry
            tf_prev, niter_prev = process(i, 0, tf_prev, niter_prev)
            tf_prev, niter_prev = process(i, 1, tf_prev, niter_prev)
            return tf_prev, niter_prev

        lax.fori_loop(0, rows_per // 2, row_pair,
                      (jnp.full((L,), 1.0, jnp.float32), jnp.int32(0)))
        pltpu.make_async_copy(zbuf, out_hbm.at[base], so).wait()

    return topk_mask


def kernel(x):
    nrows, ncols = x.shape
    return _build(nrows, ncols)(x)


# speedup vs baseline: 1.0757x; 1.0757x over previous
import functools

import jax
import jax.numpy as jnp
from jax import lax
from jax.experimental import pallas as pl
from jax.experimental.pallas import tpu as pltpu
from jax.experimental.pallas import tpu_sc as plsc

L = 16
NC = 2
NS = 16
NW = NC * NS
GRP = 16
UB = 8
US = 4
DEPTH = 64
CROWS = DEPTH + 2 * UB
KTOP = 64
MINNORM = 1.1754944e-38


@functools.lru_cache(maxsize=None)
def _build(nrows, ncols):
    assert ncols % (L * GRP) == 0 and nrows % (2 * NW) == 0
    nch = ncols // L
    ngrp = nch // GRP
    rows_per = nrows // NW

    mesh = plsc.VectorSubcoreMesh(
        core_axis_name="c", subcore_axis_name="s",
        num_cores=NC, num_subcores=NS)

    @functools.partial(
        pl.kernel,
        out_type=jax.ShapeDtypeStruct((nrows, ncols), jnp.float32),
        mesh=mesh,
        compiler_params=pltpu.CompilerParams(needs_layout_passes=False),
        scratch_types=[
            pltpu.VMEM((ncols,), jnp.float32),
            pltpu.VMEM((ncols,), jnp.float32),
            pltpu.VMEM((ncols,), jnp.float32),
            pltpu.VMEM((CROWS * L,), jnp.int32),
            pltpu.VMEM((CROWS * L,), jnp.int32),
            pltpu.VMEM((CROWS * L,), jnp.float32),
            pltpu.VMEM((CROWS * L,), jnp.float32),
            pltpu.SemaphoreType.DMA,
            pltpu.SemaphoreType.DMA,
            pltpu.SemaphoreType.DMA,
        ],
    )
    def topk_mask(x_hbm, out_hbm, rb0, rb1, zbuf, cc0, cc1, cv0, cv1,
                  si0, si1, so):
        wid = lax.axis_index("s") * NC + lax.axis_index("c")
        base = wid * rows_per
        lanes = lax.iota(jnp.int32, L)
        zf = jnp.zeros((L,), jnp.float32)
        onei = jnp.full((L,), 1, jnp.int32)
        inc16 = jnp.full((L,), 16, jnp.int32)
        zi = jnp.zeros((L,), jnp.int32)
        lane15 = jnp.full((L,), 15, jnp.int32)
        ktopv = jnp.full((L,), KTOP, jnp.int32)
        minnv = jnp.full((L,), MINNORM, jnp.float32)
        posclamp = lanes + (DEPTH - 1) * L

        rbufs = (rb0, rb1)
        ccols = (cc0, cc1)
        cvals = (cv0, cv1)
        sins = (si0, si1)

        def z0_body(g, _):
            for j in range(UB):
                zbuf[pl.ds((g * UB + j) * L, L)] = zf
            return 0

        lax.fori_loop(0, nch // UB, z0_body, 0)
        for j in range(CROWS):
            cc0[pl.ds(j * L, L)] = zi
            cc1[pl.ds(j * L, L)] = zi

        pltpu.async_copy(x_hbm.at[base], rb0, si0)
        pltpu.async_copy(x_hbm.at[base + 1], rb1, si1)

        def process(i, b, tf_prev, niter_prev):
            r = base + 2 * i + b
            rowbuf = rbufs[b]
            ccol = ccols[b]
            cval = cvals[b]
            sin = sins[b]
            nccol = ccols[1 - b]
            ncval = cvals[1 - b]

            pltpu.make_async_copy(x_hbm.at[base], rowbuf, sin).wait()

            def grp_body(g, carry):
                m0, m1, m2, m3 = carry
                gmax = zf
                for j in range(GRP):
                    gmax = jnp.maximum(gmax, rowbuf[pl.ds((g * GRP + j) * L, L)])
                t0 = jnp.maximum(m0, gmax)
                b0 = jnp.minimum(m0, gmax)
                t1 = jnp.maximum(m1, b0)
                b1 = jnp.minimum(m1, b0)
                t2 = jnp.maximum(m2, b1)
                b2 = jnp.minimum(m2, b1)
                t3 = jnp.maximum(m3, b2)
                return t0, t1, t2, t3

            _, _, _, m3 = lax.fori_loop(0, ngrp, grp_body, (zf, zf, zf, zf))
            mv0 = jnp.maximum(m3, minnv)
            mv = jnp.negative(plsc.cummax(jnp.negative(mv0))).at[lane15].get(
                mode="promise_in_bounds")

            def b_body(gb, carry):
                off, colv = carry
                vs = [rowbuf[pl.ds((gb * UB + j) * L, L)] for j in range(UB)]
                keeps = [v >= mv for v in vs]
                sels = [jnp.where(k, inc16, zi) for k in keeps]
                cols = [colv + j * L for j in range(UB)]
                for j in range(UB):
                    plsc.store_scatter(ccol, [off], cols[j], mask=keeps[j])
                    off = off + sels[j]
                return jnp.minimum(off, posclamp), colv + UB * L

            off, _ = lax.fori_loop(0, nch // UB, b_body, (lanes, lanes))
            cnt_vec = lax.shift_right_logical(off - lanes, 4)
            nrows_c = jnp.minimum(jnp.max(off) >> 4, DEPTH - 1) + 1
            niter = (nrows_c + (US - 1)) // US

            def g_body(j, rowv):
                colvs = [ccol[pl.ds((j * US + u) * L, L)] for u in range(US)]
                gs = [plsc.load_gather(rowbuf, [c]) for c in colvs]
                for u in range(US):
                    valid = (rowv + u) < cnt_vec
                    cval[pl.ds((j * US + u) * L, L)] = jnp.where(valid, gs[u], 0.0)
                return rowv + US

            lax.fori_loop(0, niter, g_body, zi)

            @pl.when(i < rows_per // 2 - 1)
            def _():
                pltpu.async_copy(x_hbm.at[r + 2], rowbuf, sin)

            def bit_body(bb, carry):
                tbv, bitv = carry
                pv = tbv | bitv
                pfv = plsc.bitcast(pv, jnp.float32)

                def cnt_body(j, acc):
                    cvs = [cval[pl.ds((j * US + u) * L, L)] for u in range(US)]
                    for cv in cvs:
                        acc = acc + jnp.where(cv >= pfv, onei, zi)
                    return acc

                acc = lax.fori_loop(0, niter, cnt_body, zi)
                tot = plsc.cumsum(acc).at[lane15].get(mode="promise_in_bounds")
                tbv = jnp.where(tot >= ktopv, pv, tbv)
                return tbv, lax.shift_right_logical(bitv, 1)

            tbv, _ = lax.fori_loop(
                0, 31, bit_body,
                (zi, jnp.full((L,), 1 << 30, jnp.int32)))
            tfv = plsc.bitcast(jnp.maximum(tbv, onei), jnp.float32)

            if b == 0:
                @pl.when(i >= 1)
                def _():
                    pltpu.make_async_copy(zbuf, out_hbm.at[base], so).wait()
            else:
                pltpu.make_async_copy(zbuf, out_hbm.at[base], so).wait()

            def uz_body(j, _):
                vals = [ncval[pl.ds((j * US + u) * L, L)] for u in range(US)]
                cols = [nccol[pl.ds((j * US + u) * L, L)] for u in range(US)]
                for u in range(US):
                    plsc.store_scatter(zbuf, [cols[u]], zf,
                                       mask=vals[u] >= tf_prev)
                return 0

            lax.fori_loop(0, niter_prev, uz_body, 0)

            def sc_body(j, _):
                vals = [cval[pl.ds((j * US + u) * L, L)] for u in range(US)]
                cols = [ccol[pl.ds((j * US + u) * L, L)] for u in range(US)]
                for u in range(US):
                    plsc.store_scatter(zbuf, [cols[u]], vals[u],
                                       mask=vals[u] >= tfv)
                return 0

            lax.fori_loop(0, niter, sc_body, 0)
            pltpu.async_copy(zbuf, out_hbm.at[r], so)
            return tfv, niter

        def row_pair(i, carry):
            tf_prev, niter_prev = carry
            tf_prev, niter_prev = process(i, 0, tf_prev, niter_prev)
            tf_prev, niter_prev = process(i, 1, tf_prev, niter_prev)
            return tf_prev, niter_prev

        lax.fori_loop(0, rows_per // 2, row_pair,
                      (jnp.full((L,), 1.0, jnp.float32), jnp.int32(0)))
        pltpu.make_async_copy(zbuf, out_hbm.at[base], so).wait()

    return topk_mask


def kernel(x):
    nrows, ncols = x.shape
    return _build(nrows, ncols)(x)
